# baseline (speedup 1.0000x reference)
import functools

import numpy as np
import jax
import jax.numpy as jnp
from jax.experimental import pallas as pl
from jax.experimental.pallas import tpu as pltpu


def _fused_kernel(x0_ref, x1_ref, x2_ref, x3_ref, g_ref, w_ref,
                  gamma_ref, beta_ref, o_ref,
                  conv_sc, stats_sc, scale_sc, shift_sc,
                  st0, st1, st2, st3, sem0, sem1, sem2, sem3,
                  *, p1_steps, p2_steps, bpc, cin, n, rstreams, wstreams,
                  bh, count, eps):
    i = pl.program_id(0)
    srcs = (x0_ref, x1_ref, x2_ref, x3_ref)[:rstreams]
    stages = (st0, st1, st2, st3)[:wstreams]
    sems = (sem0, sem1, sem2, sem3)[:wstreams]
    rchunk = n // rstreams
    wchunk = n // wstreams

    @pl.when(i < p1_steps)
    def _conv_phase():
        @pl.when(i == 0)
        def _init():
            stats_sc[...] = jnp.zeros_like(stats_sc)

        v_all = jnp.concatenate(
            [r[...].reshape(r.shape[0] * r.shape[1], r.shape[2])
             for r in srcs], axis=0)
        v_all = jnp.maximum(v_all, 0.0)
        p_all = jnp.dot(v_all.astype(jnp.bfloat16), g_ref[...],
                        preferred_element_type=jnp.float32)
        s = p_all.shape[1] // 2
        nb = rstreams * bpc
        pv_all = jnp.concatenate(
            [jnp.concatenate([p_all[b * cin:(b + 1) * cin, :s],
                              p_all[b * cin:(b + 1) * cin, s:]], axis=0)
             for b in range(nb)], axis=1)
        y_all = jnp.dot(w_ref[...], pv_all,
                        preferred_element_type=jnp.float32)
        for k in range(rstreams):
            for b in range(bpc):
                idx = k * rchunk + i * bpc + b
                j = k * bpc + b
                conv_sc[idx] = y_all[:, j * s:(j + 1) * s]
        ycat = jnp.concatenate([y_all, y_all * y_all], axis=0)
        ones = jnp.ones((8, nb * s), jnp.float32)
        stats_sc[...] += jax.lax.dot_general(
            ones, ycat, dimension_numbers=(((1,), (1,)), ((), ())),
            preferred_element_type=jnp.float32)

    @pl.when(i == p1_steps)
    def _fold():
        c = gamma_ref.shape[1]
        s = scale_sc.shape[1]
        row = stats_sc[0:1, :]
        inv_n = 1.0 / count
        mean = row[:, :c] * inv_n
        var = row[:, c:] * inv_n - mean * mean
        scale = gamma_ref[...] * jax.lax.rsqrt(var + eps)
        shift = beta_ref[...] - mean * scale
        ones = jnp.ones((1, s), jnp.float32)
        dn = (((0,), (0,)), ((), ()))
        scale_sc[...] = jax.lax.dot_general(
            scale, ones, dimension_numbers=dn,
            preferred_element_type=jnp.float32)
        shift_sc[...] = jax.lax.dot_general(
            shift, ones, dimension_numbers=dn,
            preferred_element_type=jnp.float32)

    @pl.when(i >= p1_steps)
    def _apply_phase():
        j = i - p1_steps
        buf = jax.lax.rem(j, 2)

        def _cp(k, step):
            sl = jax.lax.rem(step, 2)
            return pltpu.make_async_copy(
                stages[k].at[sl],
                o_ref.at[pl.ds(k * wchunk + step * bh, bh)],
                sems[k].at[sl])

        @pl.when(j >= 2)
        def _reclaim():
            for k in range(wstreams):
                _cp(k, j - 2).wait()

        for k in range(wstreams):
            for b in range(bh):
                stages[k][buf, b] = (conv_sc[k * wchunk + j * bh + b]
                                     * scale_sc[...] + shift_sc[...])
        for k in range(wstreams):
            _cp(k, j).start()

        @pl.when(j == p2_steps - 1)
        def _drain():
            for k in range(wstreams):
                if p2_steps > 1:
                    _cp(k, j - 1).wait()
                _cp(k, j).wait()


def kernel(x_nchw, w1, b1, w2, b2, gamma, beta, *, eps=1e-5):
    n, cin, h, w = x_nchw.shape
    half = w1.shape[0]
    cout = 2 * half
    oh, ow = h // 2, w // 2
    s = oh * ow
    hw = h * w
    rows = n * s

    x_flat = x_nchw.astype(jnp.float32).reshape(n, cin, hw)

    jj = np.arange(s)
    r_, q_ = jj // ow, jj % ow
    g_np = np.zeros((hw, 2 * s), np.float32)
    g_np[(2 * r_) * w + 2 * q_, jj] = 1.0
    g_np[(2 * r_ + 1) * w + (2 * q_ + 1), s + jj] = 1.0
    g = jnp.asarray(g_np).astype(jnp.bfloat16)

    w_bd = jnp.concatenate(
        [jnp.concatenate([w1.astype(jnp.float32),
                          jnp.zeros((half, cin), jnp.float32)], axis=1),
         jnp.concatenate([jnp.zeros((half, cin), jnp.float32),
                          w2.astype(jnp.float32)], axis=1)], axis=0)
    del b1, b2
    g_row = gamma.astype(jnp.float32).reshape(1, cout)
    beta_row = beta.astype(jnp.float32).reshape(1, cout)

    if n % 64 == 0:
        rstreams, bpc, wstreams, bh = 4, 4, 4, 8
    elif n % 32 == 0:
        rstreams, bpc, wstreams, bh = 4, 2, 4, 4
    elif n % 16 == 0:
        rstreams, bpc, wstreams, bh = 4, 1, 4, 2
    elif n % 8 == 0:
        rstreams, bpc, wstreams, bh = 2, 2, 2, 4
    elif n % 4 == 0:
        rstreams, bpc, wstreams, bh = 2, 1, 2, 1
    else:
        rstreams, bpc, wstreams, bh = 1, 1, 1, 1
    p1_steps = n // (rstreams * bpc)
    p2_steps = n // (wstreams * bh)

    body = functools.partial(
        _fused_kernel, p1_steps=p1_steps, p2_steps=p2_steps, bpc=bpc,
        cin=cin, n=n, rstreams=rstreams, wstreams=wstreams, bh=bh,
        count=float(rows), eps=eps)

    def _xspec(k):
        if k < rstreams:
            off = k * p1_steps
            return pl.BlockSpec(
                (bpc, cin, hw),
                lambda i, off=off: (jnp.minimum(i, p1_steps - 1) + off, 0, 0))
        return pl.BlockSpec((bpc, cin, hw), lambda i: (0, 0, 0))

    in_specs = [_xspec(0), _xspec(1), _xspec(2), _xspec(3),
                pl.BlockSpec((hw, 2 * s), lambda i: (0, 0)),
                pl.BlockSpec((cout, 2 * cin), lambda i: (0, 0)),
                pl.BlockSpec((1, cout), lambda i: (0, 0)),
                pl.BlockSpec((1, cout), lambda i: (0, 0))]

    out = pl.pallas_call(
        body,
        grid=(p1_steps + p2_steps,),
        in_specs=in_specs,
        out_specs=pl.BlockSpec(memory_space=pl.ANY),
        out_shape=jax.ShapeDtypeStruct((n, cout, s), jnp.float32),
        scratch_shapes=[
            pltpu.VMEM((n, cout, s), jnp.float32),
            pltpu.VMEM((8, 2 * cout), jnp.float32),
            pltpu.VMEM((cout, s), jnp.float32),
            pltpu.VMEM((cout, s), jnp.float32),
            pltpu.VMEM((2, bh, cout, s), jnp.float32),
            pltpu.VMEM((2, bh, cout, s), jnp.float32),
            pltpu.VMEM((2, bh, cout, s), jnp.float32),
            pltpu.VMEM((2, bh, cout, s), jnp.float32),
            pltpu.SemaphoreType.DMA((2,)),
            pltpu.SemaphoreType.DMA((2,)),
            pltpu.SemaphoreType.DMA((2,)),
            pltpu.SemaphoreType.DMA((2,)),
        ],
        compiler_params=pltpu.CompilerParams(
            dimension_semantics=("arbitrary",),
            vmem_limit_bytes=100 * 1024 * 1024,
        ),
        cost_estimate=pl.CostEstimate(
            flops=2 * rows * (2 * cin) * cout + 2 * n * cin * hw * 2 * s
            + 4 * rows * cout,
            transcendentals=cout,
            bytes_accessed=4 * (n * cin * hw + n * cout * s) + hw * 2 * s * 2,
        ),
    )(x_flat, x_flat, x_flat, x_flat, g, w_bd, g_row, beta_row)

    return out.reshape(n, cout, oh, ow)


# speedup vs baseline: 1.0168x; 1.0168x over previous
import functools

import numpy as np
import jax
import jax.numpy as jnp
from jax.experimental import pallas as pl
from jax.experimental.pallas import tpu as pltpu


def _fused_kernel(x0_ref, x1_ref, x2_ref, x3_ref, x4_ref, x5_ref, x6_ref,
                  x7_ref, g_ref, w_ref, gamma_ref, beta_ref, o_ref,
                  conv_sc, stats_sc, scale_sc, shift_sc,
                  st0, st1, st2, st3, st4, st5, st6, st7,
                  sem0, sem1, sem2, sem3, sem4, sem5, sem6, sem7,
                  *, p1_steps, p2_steps, bpc, cin, n, rstreams, wstreams,
                  bh, count, eps):
    i = pl.program_id(0)
    srcs = (x0_ref, x1_ref, x2_ref, x3_ref, x4_ref, x5_ref, x6_ref,
            x7_ref)[:rstreams]
    stages = (st0, st1, st2, st3, st4, st5, st6, st7)[:wstreams]
    sems = (sem0, sem1, sem2, sem3, sem4, sem5, sem6, sem7)[:wstreams]
    rchunk = n // rstreams
    wchunk = n // wstreams

    @pl.when(i < p1_steps)
    def _conv_phase():
        @pl.when(i == 0)
        def _init():
            stats_sc[...] = jnp.zeros_like(stats_sc)

        v_all = jnp.concatenate(
            [r[...].reshape(r.shape[0] * r.shape[1], r.shape[2])
             for r in srcs], axis=0)
        v_all = jnp.maximum(v_all, 0.0)
        p_all = jnp.dot(v_all.astype(jnp.bfloat16), g_ref[...],
                        preferred_element_type=jnp.float32)
        s = p_all.shape[1] // 2
        nb = rstreams * bpc
        pv_all = jnp.concatenate(
            [jnp.concatenate([p_all[b * cin:(b + 1) * cin, :s],
                              p_all[b * cin:(b + 1) * cin, s:]], axis=0)
             for b in range(nb)], axis=1)
        y_all = jnp.dot(w_ref[...], pv_all,
                        preferred_element_type=jnp.float32)
        for k in range(rstreams):
            for b in range(bpc):
                idx = k * rchunk + i * bpc + b
                j = k * bpc + b
                conv_sc[idx] = y_all[:, j * s:(j + 1) * s]
        ycat = jnp.concatenate([y_all, y_all * y_all], axis=0)
        ones = jnp.ones((8, nb * s), jnp.float32)
        stats_sc[...] += jax.lax.dot_general(
            ones, ycat, dimension_numbers=(((1,), (1,)), ((), ())),
            preferred_element_type=jnp.float32)

    @pl.when(i == p1_steps)
    def _fold():
        c = gamma_ref.shape[1]
        s = scale_sc.shape[1]
        row = stats_sc[0:1, :]
        inv_n = 1.0 / count
        mean = row[:, :c] * inv_n
        var = row[:, c:] * inv_n - mean * mean
        scale = gamma_ref[...] * jax.lax.rsqrt(var + eps)
        shift = beta_ref[...] - mean * scale
        ones = jnp.ones((1, s), jnp.float32)
        dn = (((0,), (0,)), ((), ()))
        scale_sc[...] = jax.lax.dot_general(
            scale, ones, dimension_numbers=dn,
            preferred_element_type=jnp.float32)
        shift_sc[...] = jax.lax.dot_general(
            shift, ones, dimension_numbers=dn,
            preferred_element_type=jnp.float32)

    @pl.when(i >= p1_steps)
    def _apply_phase():
        j = i - p1_steps
        buf = jax.lax.rem(j, 2)

        def _cp(k, step):
            sl = jax.lax.rem(step, 2)
            return pltpu.make_async_copy(
                stages[k].at[sl],
                o_ref.at[pl.ds(k * wchunk + step * bh, bh)],
                sems[k].at[sl])

        @pl.when(j >= 2)
        def _reclaim():
            for k in range(wstreams):
                _cp(k, j - 2).wait()

        for k in range(wstreams):
            for b in range(bh):
                stages[k][buf, b] = (conv_sc[k * wchunk + j * bh + b]
                                     * scale_sc[...] + shift_sc[...])
        for k in range(wstreams):
            _cp(k, j).start()

        @pl.when(j == p2_steps - 1)
        def _drain():
            for k in range(wstreams):
                if p2_steps > 1:
                    _cp(k, j - 1).wait()
                _cp(k, j).wait()


def kernel(x_nchw, w1, b1, w2, b2, gamma, beta, *, eps=1e-5):
    n, cin, h, w = x_nchw.shape
    half = w1.shape[0]
    cout = 2 * half
    oh, ow = h // 2, w // 2
    s = oh * ow
    hw = h * w
    rows = n * s

    x_flat = x_nchw.astype(jnp.float32).reshape(n, cin, hw)

    jj = np.arange(s)
    r_, q_ = jj // ow, jj % ow
    g_np = np.zeros((hw, 2 * s), np.float32)
    g_np[(2 * r_) * w + 2 * q_, jj] = 1.0
    g_np[(2 * r_ + 1) * w + (2 * q_ + 1), s + jj] = 1.0
    g = jnp.asarray(g_np).astype(jnp.bfloat16)

    w_bd = jnp.concatenate(
        [jnp.concatenate([w1.astype(jnp.float32),
                          jnp.zeros((half, cin), jnp.float32)], axis=1),
         jnp.concatenate([jnp.zeros((half, cin), jnp.float32),
                          w2.astype(jnp.float32)], axis=1)], axis=0)
    del b1, b2
    g_row = gamma.astype(jnp.float32).reshape(1, cout)
    beta_row = beta.astype(jnp.float32).reshape(1, cout)

    if n % 16 == 0:
        rstreams, bpc, wstreams, bh = 8, 1, 8, 2
    elif n % 8 == 0:
        rstreams, bpc, wstreams, bh = 2, 2, 2, 4
    elif n % 4 == 0:
        rstreams, bpc, wstreams, bh = 2, 1, 2, 1
    else:
        rstreams, bpc, wstreams, bh = 1, 1, 1, 1
    p1_steps = n // (rstreams * bpc)
    p2_steps = n // (wstreams * bh)

    body = functools.partial(
        _fused_kernel, p1_steps=p1_steps, p2_steps=p2_steps, bpc=bpc,
        cin=cin, n=n, rstreams=rstreams, wstreams=wstreams, bh=bh,
        count=float(rows), eps=eps)

    def _xspec(k):
        if k < rstreams:
            off = k * p1_steps
            return pl.BlockSpec(
                (bpc, cin, hw),
                lambda i, off=off: (jnp.minimum(i, p1_steps - 1) + off, 0, 0))
        return pl.BlockSpec((bpc, cin, hw), lambda i: (0, 0, 0))

    in_specs = [_xspec(k) for k in range(8)] + [
                pl.BlockSpec((hw, 2 * s), lambda i: (0, 0)),
                pl.BlockSpec((cout, 2 * cin), lambda i: (0, 0)),
                pl.BlockSpec((1, cout), lambda i: (0, 0)),
                pl.BlockSpec((1, cout), lambda i: (0, 0))]

    out = pl.pallas_call(
        body,
        grid=(p1_steps + p2_steps,),
        in_specs=in_specs,
        out_specs=pl.BlockSpec(memory_space=pl.ANY),
        out_shape=jax.ShapeDtypeStruct((n, cout, s), jnp.float32),
        scratch_shapes=[
            pltpu.VMEM((n, cout, s), jnp.float32),
            pltpu.VMEM((8, 2 * cout), jnp.float32),
            pltpu.VMEM((cout, s), jnp.float32),
            pltpu.VMEM((cout, s), jnp.float32),
        ] + [pltpu.VMEM((2, bh, cout, s), jnp.float32)] * 8
        + [pltpu.SemaphoreType.DMA((2,))] * 8,
        compiler_params=pltpu.CompilerParams(
            dimension_semantics=("arbitrary",),
            vmem_limit_bytes=100 * 1024 * 1024,
        ),
        cost_estimate=pl.CostEstimate(
            flops=2 * rows * (2 * cin) * cout + 2 * n * cin * hw * 2 * s
            + 4 * rows * cout,
            transcendentals=cout,
            bytes_accessed=4 * (n * cin * hw + n * cout * s) + hw * 2 * s * 2,
        ),
    )(*([x_flat] * 8), g, w_bd, g_row, beta_row)

    return out.reshape(n, cout, oh, ow)


# speedup vs baseline: 1.0211x; 1.0042x over previous
import functools

import numpy as np
import jax
import jax.numpy as jnp
from jax.experimental import pallas as pl
from jax.experimental.pallas import tpu as pltpu


def _fused_kernel(x0_ref, x1_ref, x2_ref, x3_ref, g_ref, w_ref,
                  gamma_ref, beta_ref, o_ref,
                  conv_sc, stats_sc, scale_sc, shift_sc,
                  st0, st1, st2, st3, sem0, sem1, sem2, sem3,
                  *, p1_steps, p2_steps, bpc, cin, n, rstreams, wstreams,
                  bh, count, eps):
    i = pl.program_id(0)
    srcs = (x0_ref, x1_ref, x2_ref, x3_ref)[:rstreams]
    stages = (st0, st1, st2, st3)[:wstreams]
    sems = (sem0, sem1, sem2, sem3)[:wstreams]
    rchunk = n // rstreams
    wchunk = n // wstreams

    @pl.when(i < p1_steps)
    def _conv_phase():
        @pl.when(i == 0)
        def _init():
            stats_sc[...] = jnp.zeros_like(stats_sc)

        v_all = jnp.concatenate(
            [r[...].reshape(r.shape[0] * r.shape[1], r.shape[2])
             for r in srcs], axis=0)
        v_all = jnp.maximum(v_all, 0.0)
        p_all = jnp.dot(v_all.astype(jnp.bfloat16), g_ref[...],
                        preferred_element_type=jnp.float32)
        s = p_all.shape[1] // 2
        nb = rstreams * bpc
        pv_all = jnp.concatenate(
            [jnp.concatenate([p_all[b * cin:(b + 1) * cin, :s],
                              p_all[b * cin:(b + 1) * cin, s:]], axis=0)
             for b in range(nb)], axis=1)
        y_all = jnp.dot(w_ref[...], pv_all,
                        preferred_element_type=jnp.float32)
        for k in range(rstreams):
            for b in range(bpc):
                idx = k * rchunk + i * bpc + b
                j = k * bpc + b
                conv_sc[idx] = y_all[:, j * s:(j + 1) * s]
        ycat = jnp.concatenate([y_all, y_all * y_all], axis=0)
        ones = jnp.ones((8, nb * s), jnp.float32)
        stats_sc[...] += jax.lax.dot_general(
            ones, ycat, dimension_numbers=(((1,), (1,)), ((), ())),
            preferred_element_type=jnp.float32)

    @pl.when(i == p1_steps)
    def _fold():
        c = gamma_ref.shape[1]
        s = scale_sc.shape[1]
        row = stats_sc[0:1, :]
        inv_n = 1.0 / count
        mean = row[:, :c] * inv_n
        var = row[:, c:] * inv_n - mean * mean
        scale = gamma_ref[...] * jax.lax.rsqrt(var + eps)
        shift = beta_ref[...] - mean * scale
        ones = jnp.ones((1, s), jnp.float32)
        dn = (((0,), (0,)), ((), ()))
        scale_sc[...] = jax.lax.dot_general(
            scale, ones, dimension_numbers=dn,
            preferred_element_type=jnp.float32)
        shift_sc[...] = jax.lax.dot_general(
            shift, ones, dimension_numbers=dn,
            preferred_element_type=jnp.float32)

    @pl.when(i >= p1_steps)
    def _apply_phase():
        j = i - p1_steps
        buf = jax.lax.rem(j, 2)

        def _cp(k, step):
            sl = jax.lax.rem(step, 2)
            return pltpu.make_async_copy(
                stages[k].at[sl],
                o_ref.at[pl.ds(k * wchunk + step * bh, bh)],
                sems[k].at[sl])

        @pl.when(j >= 2)
        def _reclaim():
            for k in range(wstreams):
                _cp(k, j - 2).wait()

        for k in range(wstreams):
            for b in range(bh):
                stages[k][buf, b] = (conv_sc[k * wchunk + j * bh + b]
                                     * scale_sc[...] + shift_sc[...])
        for k in range(wstreams):
            _cp(k, j).start()

        @pl.when(j == p2_steps - 1)
        def _drain():
            for k in range(wstreams):
                if p2_steps > 1:
                    _cp(k, j - 1).wait()
                _cp(k, j).wait()


def kernel(x_nchw, w1, b1, w2, b2, gamma, beta, *, eps=1e-5):
    n, cin, h, w = x_nchw.shape
    half = w1.shape[0]
    cout = 2 * half
    oh, ow = h // 2, w // 2
    s = oh * ow
    hw = h * w
    rows = n * s

    x_flat = x_nchw.astype(jnp.float32).reshape(n, cin, hw)

    jj = np.arange(s)
    r_, q_ = jj // ow, jj % ow
    g_np = np.zeros((hw, 2 * s), np.float32)
    g_np[(2 * r_) * w + 2 * q_, jj] = 1.0
    g_np[(2 * r_ + 1) * w + (2 * q_ + 1), s + jj] = 1.0
    g = jnp.asarray(g_np).astype(jnp.bfloat16)

    w_bd = jnp.concatenate(
        [jnp.concatenate([w1.astype(jnp.float32),
                          jnp.zeros((half, cin), jnp.float32)], axis=1),
         jnp.concatenate([jnp.zeros((half, cin), jnp.float32),
                          w2.astype(jnp.float32)], axis=1)], axis=0)
    del b1, b2
    g_row = gamma.astype(jnp.float32).reshape(1, cout)
    beta_row = beta.astype(jnp.float32).reshape(1, cout)

    if n % 32 == 0:
        rstreams, bpc, wstreams, bh = 4, 2, 4, 4
    elif n % 16 == 0:
        rstreams, bpc, wstreams, bh = 4, 1, 4, 2
    elif n % 8 == 0:
        rstreams, bpc, wstreams, bh = 2, 2, 2, 4
    elif n % 4 == 0:
        rstreams, bpc, wstreams, bh = 2, 1, 2, 1
    else:
        rstreams, bpc, wstreams, bh = 1, 1, 1, 1
    p1_steps = n // (rstreams * bpc)
    p2_steps = n // (wstreams * bh)

    body = functools.partial(
        _fused_kernel, p1_steps=p1_steps, p2_steps=p2_steps, bpc=bpc,
        cin=cin, n=n, rstreams=rstreams, wstreams=wstreams, bh=bh,
        count=float(rows), eps=eps)

    def _xspec(k):
        if k < rstreams:
            off = k * p1_steps
            return pl.BlockSpec(
                (bpc, cin, hw),
                lambda i, off=off: (jnp.minimum(i, p1_steps - 1) + off, 0, 0))
        return pl.BlockSpec((bpc, cin, hw), lambda i: (0, 0, 0))

    in_specs = [_xspec(0), _xspec(1), _xspec(2), _xspec(3),
                pl.BlockSpec((hw, 2 * s), lambda i: (0, 0)),
                pl.BlockSpec((cout, 2 * cin), lambda i: (0, 0)),
                pl.BlockSpec((1, cout), lambda i: (0, 0)),
                pl.BlockSpec((1, cout), lambda i: (0, 0))]

    out = pl.pallas_call(
        body,
        grid=(p1_steps + p2_steps,),
        in_specs=in_specs,
        out_specs=pl.BlockSpec(memory_space=pl.ANY),
        out_shape=jax.ShapeDtypeStruct((n, cout, s), jnp.float32),
        scratch_shapes=[
            pltpu.VMEM((n, cout, s), jnp.float32),
            pltpu.VMEM((8, 2 * cout), jnp.float32),
            pltpu.VMEM((cout, s), jnp.float32),
            pltpu.VMEM((cout, s), jnp.float32),
            pltpu.VMEM((2, bh, cout, s), jnp.float32),
            pltpu.VMEM((2, bh, cout, s), jnp.float32),
            pltpu.VMEM((2, bh, cout, s), jnp.float32),
            pltpu.VMEM((2, bh, cout, s), jnp.float32),
            pltpu.SemaphoreType.DMA((2,)),
            pltpu.SemaphoreType.DMA((2,)),
            pltpu.SemaphoreType.DMA((2,)),
            pltpu.SemaphoreType.DMA((2,)),
        ],
        compiler_params=pltpu.CompilerParams(
            dimension_semantics=("arbitrary",),
            vmem_limit_bytes=100 * 1024 * 1024,
        ),
        cost_estimate=pl.CostEstimate(
            flops=2 * rows * (2 * cin) * cout + 2 * n * cin * hw * 2 * s
            + 4 * rows * cout,
            transcendentals=cout,
            bytes_accessed=4 * (n * cin * hw + n * cout * s) + hw * 2 * s * 2,
        ),
    )(x_flat, x_flat, x_flat, x_flat, g, w_bd, g_row, beta_row)

    return out.reshape(n, cout, oh, ow)
